# baseline (speedup 1.0000x reference)
import functools

import jax
import jax.numpy as jnp
from jax import lax
from jax.experimental import pallas as pl
from jax.experimental.pallas import tpu as pltpu
from jax.experimental.pallas import tpu_sc as plsc

_BM = 256
_BR = 512


def _router(xf, w_router):
    t, d = xf.shape
    e_num = w_router.shape[1]
    nl = 128
    w_pad = jnp.pad(w_router, ((0, 0), (0, nl - e_num)))
    n_tiles = t // _BR

    def body(x_ref, w_ref, idx_ref, gat_ref, rkp_ref, stats_ref, carry_ref):
        i = pl.program_id(0)
        br = x_ref.shape[0]

        @pl.when(i == 0)
        def _():
            carry_ref[...] = jnp.zeros_like(carry_ref)
            stats_ref[...] = jnp.zeros_like(stats_ref)

        lane = jax.lax.broadcasted_iota(jnp.int32, (br, nl), 1)
        evalid = lane < e_num
        l = jnp.dot(x_ref[...], w_ref[...], preferred_element_type=jnp.float32)
        l = jnp.where(evalid, l, -1e30)
        m1 = jnp.max(l, axis=1, keepdims=True)
        i1 = jnp.min(jnp.where(l == m1, lane, nl), axis=1, keepdims=True)
        lm = jnp.where(lane == i1, -jnp.inf, l)
        m2 = jnp.max(lm, axis=1, keepdims=True)
        i2 = jnp.min(jnp.where(lm == m2, lane, nl), axis=1, keepdims=True)
        tt = jnp.exp(m2 - m1)
        s = 1.0 + tt
        g1 = 1.0 / s
        g2 = tt / s

        p = jnp.where(evalid, jnp.exp(l - m1), 0.0)
        ps = jnp.sum(p, axis=1, keepdims=True)
        probs_sum = jnp.sum(p / ps, axis=0, keepdims=True)
        lse = m1 + jnp.log(ps)
        lsesq = jnp.broadcast_to(
            jnp.sum(lse * lse, axis=0, keepdims=True)[:, :1], (1, nl)
        )

        oh1 = (lane == i1).astype(jnp.float32)
        oh2 = (lane == i2).astype(jnp.float32)
        u = oh1 + oh2
        freq = jnp.sum(oh1 + oh2 * (g2 > 0).astype(jnp.float32), axis=0,
                       keepdims=True)
        cnt = jnp.sum(u, axis=0, keepdims=True)

        ui = u.astype(jnp.int32)
        row = jax.lax.broadcasted_iota(jnp.int32, (br, nl), 0)
        cum = ui
        sh = 1
        while sh < br:
            cum = cum + jnp.where(row >= sh, pltpu.roll(cum, sh, axis=0), 0)
            sh *= 2
        cum = cum - ui + carry_ref[...].astype(jnp.int32)
        rk1 = jnp.sum(cum * (lane == i1).astype(jnp.int32), axis=1,
                      keepdims=True)
        rk2 = jnp.sum(cum * (lane == i2).astype(jnp.int32), axis=1,
                      keepdims=True)

        idx_ref[...] = jnp.concatenate([i1, i2], axis=1)
        gat_ref[...] = jnp.concatenate([g1, g2], axis=1)
        rkp_ref[...] = jnp.concatenate([rk1, rk2], axis=1)
        zrow = jnp.zeros((1, nl), jnp.float32)
        stats_ref[...] += jnp.concatenate(
            [probs_sum, freq, lsesq, cnt, zrow, zrow, zrow, zrow], axis=0
        )
        carry_ref[...] += cnt

    return pl.pallas_call(
        body,
        grid=(n_tiles,),
        in_specs=[
            pl.BlockSpec((_BR, d), lambda i: (i, 0)),
            pl.BlockSpec((d, nl), lambda i: (0, 0)),
        ],
        out_specs=[
            pl.BlockSpec((_BR, 2), lambda i: (i, 0)),
            pl.BlockSpec((_BR, 2), lambda i: (i, 0)),
            pl.BlockSpec((_BR, 2), lambda i: (i, 0)),
            pl.BlockSpec((8, nl), lambda i: (0, 0)),
        ],
        out_shape=[
            jax.ShapeDtypeStruct((t, 2), jnp.int32),
            jax.ShapeDtypeStruct((t, 2), jnp.float32),
            jax.ShapeDtypeStruct((t, 2), jnp.int32),
            jax.ShapeDtypeStruct((8, nl), jnp.float32),
        ],
        scratch_shapes=[pltpu.VMEM((1, nl), jnp.float32)],
        compiler_params=pltpu.CompilerParams(
            dimension_semantics=("arbitrary",),
        ),
    )(xf, w_pad)


def _gmm_body(em, tm, vm, se, ee, x_ref, win_ref, wout_ref, b_ref, out_ref):
    i = pl.program_id(0)
    e = em[i]
    t = tm[i]
    valid = vm[i]
    bm, d = x_ref.shape
    h2 = win_ref.shape[2]
    h = h2 // 2

    @pl.when(valid == 1)
    def _():
        rows = t * bm + jax.lax.broadcasted_iota(jnp.int32, (bm, 1), 0)
        mask = (rows >= se[e]) & (rows < ee[e])
        hh = jnp.dot(x_ref[...], win_ref[0], preferred_element_type=jnp.float32)
        h1 = hh[:, :h]
        hg = hh[:, h:]
        act = h1 * jax.nn.sigmoid(h1) * hg
        o = jnp.dot(act, wout_ref[0], preferred_element_type=jnp.float32)
        o = o + b_ref[...]
        out_ref[...] = jnp.where(mask, o, out_ref[...])


def _grouped_ffn(x_sorted, w_in, w_out, bias, starts, ends):
    tk, d = x_sorted.shape
    e_num, _, h2 = w_in.shape
    h = h2 // 2
    m_tiles = tk // _BM
    max_steps = m_tiles + e_num - 1

    counts = ends - starts
    tile_lo = starts // _BM
    tile_hi = (ends + _BM - 1) // _BM
    ntiles = jnp.where(counts > 0, tile_hi - tile_lo, 0)
    cum = jnp.cumsum(ntiles)
    total = cum[-1]
    first_step = cum - ntiles

    steps = jnp.arange(max_steps, dtype=jnp.int32)
    e_of = jnp.searchsorted(cum, steps, side="right").astype(jnp.int32)
    valid = (steps < total).astype(jnp.int32)
    e_last = jnp.searchsorted(cum, total - 1, side="right").astype(jnp.int32)
    e_of = jnp.where(valid == 1, jnp.minimum(e_of, e_num - 1), e_last)
    t_of = jnp.where(
        valid == 1,
        tile_lo[e_of] + steps - first_step[e_of],
        m_tiles - 1,
    ).astype(jnp.int32)

    grid_spec = pltpu.PrefetchScalarGridSpec(
        num_scalar_prefetch=5,
        grid=(max_steps,),
        in_specs=[
            pl.BlockSpec((_BM, d), lambda i, em, tm, vm, se, ee: (tm[i], 0)),
            pl.BlockSpec((1, d, h2), lambda i, em, tm, vm, se, ee: (em[i], 0, 0)),
            pl.BlockSpec((1, h, d), lambda i, em, tm, vm, se, ee: (em[i], 0, 0)),
            pl.BlockSpec((1, d), lambda i, em, tm, vm, se, ee: (0, 0)),
        ],
        out_specs=pl.BlockSpec((_BM, d), lambda i, em, tm, vm, se, ee: (tm[i], 0)),
    )
    return pl.pallas_call(
        _gmm_body,
        grid_spec=grid_spec,
        out_shape=jax.ShapeDtypeStruct((tk, d), jnp.float32),
        compiler_params=pltpu.CompilerParams(
            dimension_semantics=("arbitrary",),
            vmem_limit_bytes=100 * 1024 * 1024,
        ),
    )(
        e_of,
        t_of,
        valid,
        starts.astype(jnp.int32),
        ends.astype(jnp.int32),
        x_sorted,
        w_in,
        w_out,
        bias[None, :],
    )


def _sc_dispatch_scatter(xf, re3, ro3):
    t, d = xf.shape
    tk = 2 * t
    nw, nch, ch = re3.shape
    tpw = t // nw
    mesh = plsc.VectorSubcoreMesh(core_axis_name="c", subcore_axis_name="s")

    @functools.partial(
        pl.kernel,
        mesh=mesh,
        out_type=jax.ShapeDtypeStruct((tk, d), jnp.float32),
        scratch_types=[
            pltpu.VMEM((nch, ch), jnp.int32),
            pltpu.VMEM((nch, ch), jnp.int32),
            pltpu.VMEM((ch, d), jnp.float32),
            pltpu.VMEM((ch, d), jnp.float32),
            pltpu.SemaphoreType.DMA,
            pltpu.SemaphoreType.DMA,
            pltpu.SemaphoreType.DMA,
        ],
    )
    def k(xf_hbm, re_hbm, ro_hbm, out_hbm, re_v, ro_v, buf0, buf1, seml,
          seme, semo):
        wid = lax.axis_index("s") * (nw // 16) + lax.axis_index("c")
        base = wid * tpw
        pltpu.sync_copy(re_hbm.at[wid], re_v)
        pltpu.sync_copy(ro_hbm.at[wid], ro_v)
        bufs = (buf0, buf1)
        cps = [None, None, None, None]
        for c in range(nch):
            b = bufs[c % 2]
            if c > 1:
                cps[2 * (c % 2)].wait()
                cps[2 * (c % 2) + 1].wait()
            pltpu.sync_copy(xf_hbm.at[pl.ds(base + c * ch, ch)], b)
            cps[2 * (c % 2)] = pltpu.async_copy(
                b, out_hbm.at[re_v.at[c]], seme
            )
            cps[2 * (c % 2) + 1] = pltpu.async_copy(
                b, out_hbm.at[ro_v.at[c]], semo
            )
        for c in range(max(0, nch - 2), nch):
            cps[2 * (c % 2)].wait()
            cps[2 * (c % 2) + 1].wait()

    return k(xf, re3, ro3)


def _sc_combine(out_w, re, ro, ge, go):
    tk, d = out_w.shape
    t = re.shape[0]
    info = plsc.get_sparse_core_info()
    nl = info.num_lanes
    nw = info.num_cores * info.num_subcores
    tpw = t // nw
    ch = 16
    nch = tpw // ch
    nlc = d // nl
    mesh = plsc.VectorSubcoreMesh(core_axis_name="c", subcore_axis_name="s")

    @functools.partial(
        pl.kernel,
        mesh=mesh,
        out_type=jax.ShapeDtypeStruct((t, d), jnp.float32),
        scratch_types=[
            pltpu.VMEM((tpw,), jnp.int32),
            pltpu.VMEM((tpw,), jnp.int32),
            pltpu.VMEM((tpw, 16), jnp.float32),
            pltpu.VMEM((tpw, 16), jnp.float32),
            pltpu.VMEM((ch, d), jnp.float32),
            pltpu.VMEM((ch, d), jnp.float32),
            pltpu.VMEM((ch, d), jnp.float32),
            pltpu.VMEM((ch, d), jnp.float32),
            pltpu.SemaphoreType.DMA,
            pltpu.SemaphoreType.DMA,
            pltpu.SemaphoreType.DMA,
            pltpu.SemaphoreType.DMA,
            pltpu.SemaphoreType.DMA,
            pltpu.SemaphoreType.DMA,
        ],
    )
    def k(ow_hbm, re_hbm, ro_hbm, ge_hbm, go_hbm, y_hbm, re_v, ro_v, ge_v,
          go_v, a0, a1, b0, b1, sa0, sa1, sb0, sb1, ss0, ss1):
        wid = lax.axis_index("s") * info.num_cores + lax.axis_index("c")
        base = wid * tpw
        pltpu.sync_copy(re_hbm.at[pl.ds(base, tpw)], re_v)
        pltpu.sync_copy(ro_hbm.at[pl.ds(base, tpw)], ro_v)
        pltpu.sync_copy(ge_hbm.at[pl.ds(base, tpw)], ge_v)
        pltpu.sync_copy(go_hbm.at[pl.ds(base, tpw)], go_v)
        bufa = (a0, a1)
        bufb = (b0, b1)
        sems_a = (sa0, sa1)
        sems_b = (sb0, sb1)
        sems_s = (ss0, ss1)

        def fire(c):
            s = c % 2
            return (
                pltpu.async_copy(
                    ow_hbm.at[re_v.at[pl.ds(c * ch, ch)]], bufa[s], sems_a[s]
                ),
                pltpu.async_copy(
                    ow_hbm.at[ro_v.at[pl.ds(c * ch, ch)]], bufb[s], sems_b[s]
                ),
            )

        pend = fire(0)
        stores = [None, None]
        for c in range(nch):
            s = c % 2
            if c + 1 < nch:
                if stores[(c + 1) % 2] is not None:
                    stores[(c + 1) % 2].wait()
                nxt = fire(c + 1)
            pend[0].wait()
            pend[1].wait()
            if c + 1 < nch:
                pend = nxt
            av = bufa[s]
            bv = bufb[s]

            def row_body(r, _):
                ga = ge_v[c * ch + r]
                gb = go_v[c * ch + r]

                def lane_body(l, _):
                    o = l * 4 * nl
                    for u in range(4):
                        q = o + u * nl
                        av[r, pl.ds(q, nl)] = (
                            av[r, pl.ds(q, nl)] * ga + bv[r, pl.ds(q, nl)] * gb
                        )
                    return 0

                lax.fori_loop(0, nlc // 4, lane_body, 0)
                return 0

            lax.fori_loop(0, ch, row_body, 0)
            stores[s] = pltpu.async_copy(
                av, y_hbm.at[pl.ds(base + c * ch, ch)], sems_s[s]
            )
        for s in (0, 1):
            if stores[s] is not None:
                stores[s].wait()

    return k(out_w, re, ro, ge, go)


@jax.jit
def kernel(x, w_router, w_in, w_out, bias):
    bsz, length, d = x.shape
    e_num = w_router.shape[1]
    k = 2
    xf = x.reshape(-1, d)
    t = xf.shape[0]

    top_k_indices, top_k_gates, rankp, stats = _router(xf, w_router)
    probs_sum = stats[0, :e_num]
    freq = stats[1, :e_num]
    lsesq = stats[2, 0]
    counts = stats[3, :e_num].astype(jnp.int32)
    probs_normalized = probs_sum / jnp.sum(probs_sum)
    freq_normalized = freq / jnp.sum(freq)
    switchloss = e_num * (probs_normalized * freq_normalized).sum()
    zloss = lsesq / t
    loss = switchloss + 0.1 * zloss

    flat_experts = top_k_indices.reshape(-1)
    ends = jnp.cumsum(counts).astype(jnp.int32)
    starts = ends - counts
    rank = (starts[flat_experts] + rankp.reshape(-1)).reshape(t, k)
    nw = 32
    ch_d = min(32, t // nw)
    re3 = rank[:, 0].reshape(nw, -1, ch_d)
    ro3 = rank[:, 1].reshape(nw, -1, ch_d)

    x_sorted = _sc_dispatch_scatter(xf, re3, ro3)
    out_w = _grouped_ffn(x_sorted, w_in, w_out, bias, starts, ends)

    ge_x = jnp.broadcast_to(top_k_gates[:, 0:1], (t, 16))
    go_x = jnp.broadcast_to(top_k_gates[:, 1:2], (t, 16))
    y = _sc_combine(out_w, rank[:, 0], rank[:, 1], ge_x, go_x)
    y = y.reshape(bsz, length, d)
    return (y, loss)


# speedup vs baseline: 1.0400x; 1.0400x over previous
import functools

import jax
import jax.numpy as jnp
from jax import lax
from jax.experimental import pallas as pl
from jax.experimental.pallas import tpu as pltpu
from jax.experimental.pallas import tpu_sc as plsc

_BM = 512
_BR = 512


def _router(xf, w_router):
    t, d = xf.shape
    e_num = w_router.shape[1]
    nl = 128
    w_pad = jnp.pad(w_router, ((0, 0), (0, nl - e_num)))
    n_tiles = t // _BR

    def body(x_ref, w_ref, idx_ref, ge_ref, go_ref, rkp_ref, stats_ref,
             carry_ref):
        i = pl.program_id(0)
        br = x_ref.shape[0]

        @pl.when(i == 0)
        def _():
            carry_ref[...] = jnp.zeros_like(carry_ref)
            stats_ref[...] = jnp.zeros_like(stats_ref)

        lane = jax.lax.broadcasted_iota(jnp.int32, (br, nl), 1)
        evalid = lane < e_num
        l = jnp.dot(x_ref[...], w_ref[...], preferred_element_type=jnp.float32)
        l = jnp.where(evalid, l, -1e30)
        m1 = jnp.max(l, axis=1, keepdims=True)
        i1 = jnp.min(jnp.where(l == m1, lane, nl), axis=1, keepdims=True)
        lm = jnp.where(lane == i1, -jnp.inf, l)
        m2 = jnp.max(lm, axis=1, keepdims=True)
        i2 = jnp.min(jnp.where(lm == m2, lane, nl), axis=1, keepdims=True)
        tt = jnp.exp(m2 - m1)
        s = 1.0 + tt
        g1 = 1.0 / s
        g2 = tt / s

        p = jnp.where(evalid, jnp.exp(l - m1), 0.0)
        ps = jnp.sum(p, axis=1, keepdims=True)
        probs_sum = jnp.sum(p / ps, axis=0, keepdims=True)
        lse = m1 + jnp.log(ps)
        lsesq = jnp.broadcast_to(
            jnp.sum(lse * lse, axis=0, keepdims=True)[:, :1], (1, nl)
        )

        oh1 = (lane == i1).astype(jnp.float32)
        oh2 = (lane == i2).astype(jnp.float32)
        u = oh1 + oh2
        freq = jnp.sum(oh1 + oh2 * (g2 > 0).astype(jnp.float32), axis=0,
                       keepdims=True)
        cnt = jnp.sum(u, axis=0, keepdims=True)

        ui = u.astype(jnp.int32)
        row = jax.lax.broadcasted_iota(jnp.int32, (br, nl), 0)
        cum = ui
        sh = 1
        while sh < br:
            cum = cum + jnp.where(row >= sh, pltpu.roll(cum, sh, axis=0), 0)
            sh *= 2
        cum = cum - ui + carry_ref[...].astype(jnp.int32)
        rk1 = jnp.sum(cum * (lane == i1).astype(jnp.int32), axis=1,
                      keepdims=True)
        rk2 = jnp.sum(cum * (lane == i2).astype(jnp.int32), axis=1,
                      keepdims=True)

        idx_ref[...] = jnp.concatenate([i1, i2], axis=1)
        ge_ref[...] = jnp.broadcast_to(g1, (br, 16))
        go_ref[...] = jnp.broadcast_to(g2, (br, 16))
        rkp_ref[...] = jnp.concatenate([rk1, rk2], axis=1)
        zrow = jnp.zeros((1, nl), jnp.float32)
        stats_ref[...] += jnp.concatenate(
            [probs_sum, freq, lsesq, cnt, zrow, zrow, zrow, zrow], axis=0
        )
        carry_ref[...] += cnt

    return pl.pallas_call(
        body,
        grid=(n_tiles,),
        in_specs=[
            pl.BlockSpec((_BR, d), lambda i: (i, 0)),
            pl.BlockSpec((d, nl), lambda i: (0, 0)),
        ],
        out_specs=[
            pl.BlockSpec((_BR, 2), lambda i: (i, 0)),
            pl.BlockSpec((_BR, 16), lambda i: (i, 0)),
            pl.BlockSpec((_BR, 16), lambda i: (i, 0)),
            pl.BlockSpec((_BR, 2), lambda i: (i, 0)),
            pl.BlockSpec((8, nl), lambda i: (0, 0)),
        ],
        out_shape=[
            jax.ShapeDtypeStruct((t, 2), jnp.int32),
            jax.ShapeDtypeStruct((t, 16), jnp.float32),
            jax.ShapeDtypeStruct((t, 16), jnp.float32),
            jax.ShapeDtypeStruct((t, 2), jnp.int32),
            jax.ShapeDtypeStruct((8, nl), jnp.float32),
        ],
        scratch_shapes=[pltpu.VMEM((1, nl), jnp.float32)],
        compiler_params=pltpu.CompilerParams(
            dimension_semantics=("arbitrary",),
        ),
    )(xf, w_pad)


def _gmm_body(em, tm, vm, se, ee, x_ref, win_ref, wout_ref, b_ref, out_ref):
    i = pl.program_id(0)
    e = em[i]
    t = tm[i]
    valid = vm[i]
    bm, d = x_ref.shape
    h2 = win_ref.shape[2]
    h = h2 // 2

    @pl.when(valid == 1)
    def _():
        rows = t * bm + jax.lax.broadcasted_iota(jnp.int32, (bm, 1), 0)
        mask = (rows >= se[e]) & (rows < ee[e])
        hh = jnp.dot(x_ref[...], win_ref[0], preferred_element_type=jnp.float32)
        h1 = hh[:, :h]
        hg = hh[:, h:]
        act = h1 * jax.nn.sigmoid(h1) * hg
        o = jnp.dot(act, wout_ref[0], preferred_element_type=jnp.float32)
        o = o + b_ref[...]
        out_ref[...] = jnp.where(mask, o, out_ref[...])


def _grouped_ffn(x_sorted, w_in, w_out, bias, starts, ends):
    tk, d = x_sorted.shape
    e_num, _, h2 = w_in.shape
    h = h2 // 2
    m_tiles = tk // _BM
    max_steps = m_tiles + e_num - 1

    counts = ends - starts
    tile_lo = starts // _BM
    tile_hi = (ends + _BM - 1) // _BM
    ntiles = jnp.where(counts > 0, tile_hi - tile_lo, 0)
    cum = jnp.cumsum(ntiles)
    total = cum[-1]
    first_step = cum - ntiles

    steps = jnp.arange(max_steps, dtype=jnp.int32)
    e_of = jnp.searchsorted(cum, steps, side="right").astype(jnp.int32)
    valid = (steps < total).astype(jnp.int32)
    e_last = jnp.searchsorted(cum, total - 1, side="right").astype(jnp.int32)
    e_of = jnp.where(valid == 1, jnp.minimum(e_of, e_num - 1), e_last)
    t_of = jnp.where(
        valid == 1,
        tile_lo[e_of] + steps - first_step[e_of],
        m_tiles - 1,
    ).astype(jnp.int32)

    grid_spec = pltpu.PrefetchScalarGridSpec(
        num_scalar_prefetch=5,
        grid=(max_steps,),
        in_specs=[
            pl.BlockSpec((_BM, d), lambda i, em, tm, vm, se, ee: (tm[i], 0)),
            pl.BlockSpec((1, d, h2), lambda i, em, tm, vm, se, ee: (em[i], 0, 0)),
            pl.BlockSpec((1, h, d), lambda i, em, tm, vm, se, ee: (em[i], 0, 0)),
            pl.BlockSpec((1, d), lambda i, em, tm, vm, se, ee: (0, 0)),
        ],
        out_specs=pl.BlockSpec((_BM, d), lambda i, em, tm, vm, se, ee: (tm[i], 0)),
    )
    return pl.pallas_call(
        _gmm_body,
        grid_spec=grid_spec,
        out_shape=jax.ShapeDtypeStruct((tk, d), jnp.float32),
        compiler_params=pltpu.CompilerParams(
            dimension_semantics=("arbitrary",),
            vmem_limit_bytes=100 * 1024 * 1024,
        ),
    )(
        e_of,
        t_of,
        valid,
        starts.astype(jnp.int32),
        ends.astype(jnp.int32),
        x_sorted,
        w_in,
        w_out,
        bias[None, :],
    )


def _sc_dispatch_scatter(xf, re3, ro3):
    t, d = xf.shape
    tk = 2 * t
    nw, nch, ch = re3.shape
    tpw = t // nw
    mesh = plsc.VectorSubcoreMesh(core_axis_name="c", subcore_axis_name="s")

    @functools.partial(
        pl.kernel,
        mesh=mesh,
        out_type=jax.ShapeDtypeStruct((tk, d), jnp.float32),
        scratch_types=[
            pltpu.VMEM((nch, ch), jnp.int32),
            pltpu.VMEM((nch, ch), jnp.int32),
            pltpu.VMEM((ch, d), jnp.float32),
            pltpu.VMEM((ch, d), jnp.float32),
            pltpu.SemaphoreType.DMA,
            pltpu.SemaphoreType.DMA,
            pltpu.SemaphoreType.DMA,
        ],
    )
    def k(xf_hbm, re_hbm, ro_hbm, out_hbm, re_v, ro_v, buf0, buf1, seml,
          seme, semo):
        wid = lax.axis_index("s") * (nw // 16) + lax.axis_index("c")
        base = wid * tpw
        pltpu.sync_copy(re_hbm.at[wid], re_v)
        pltpu.sync_copy(ro_hbm.at[wid], ro_v)
        bufs = (buf0, buf1)
        cps = [None, None, None, None]
        for c in range(nch):
            b = bufs[c % 2]
            if c > 1:
                cps[2 * (c % 2)].wait()
                cps[2 * (c % 2) + 1].wait()
            pltpu.sync_copy(xf_hbm.at[pl.ds(base + c * ch, ch)], b)
            cps[2 * (c % 2)] = pltpu.async_copy(
                b, out_hbm.at[re_v.at[c]], seme
            )
            cps[2 * (c % 2) + 1] = pltpu.async_copy(
                b, out_hbm.at[ro_v.at[c]], semo
            )
        for c in range(max(0, nch - 2), nch):
            cps[2 * (c % 2)].wait()
            cps[2 * (c % 2) + 1].wait()

    return k(xf, re3, ro3)


def _sc_combine(out_w, re, ro, ge, go):
    tk, d = out_w.shape
    t = re.shape[0]
    info = plsc.get_sparse_core_info()
    nl = info.num_lanes
    nw = info.num_cores * info.num_subcores
    tpw = t // nw
    ch = 16
    nch = tpw // ch
    nlc = d // nl
    mesh = plsc.VectorSubcoreMesh(core_axis_name="c", subcore_axis_name="s")

    @functools.partial(
        pl.kernel,
        mesh=mesh,
        out_type=jax.ShapeDtypeStruct((t, d), jnp.float32),
        scratch_types=[
            pltpu.VMEM((tpw,), jnp.int32),
            pltpu.VMEM((tpw,), jnp.int32),
            pltpu.VMEM((tpw, 16), jnp.float32),
            pltpu.VMEM((tpw, 16), jnp.float32),
            pltpu.VMEM((ch, d), jnp.float32),
            pltpu.VMEM((ch, d), jnp.float32),
            pltpu.VMEM((ch, d), jnp.float32),
            pltpu.VMEM((ch, d), jnp.float32),
            pltpu.SemaphoreType.DMA,
            pltpu.SemaphoreType.DMA,
            pltpu.SemaphoreType.DMA,
            pltpu.SemaphoreType.DMA,
            pltpu.SemaphoreType.DMA,
            pltpu.SemaphoreType.DMA,
        ],
    )
    def k(ow_hbm, re_hbm, ro_hbm, ge_hbm, go_hbm, y_hbm, re_v, ro_v, ge_v,
          go_v, a0, a1, b0, b1, sa0, sa1, sb0, sb1, ss0, ss1):
        wid = lax.axis_index("s") * info.num_cores + lax.axis_index("c")
        base = wid * tpw
        pltpu.sync_copy(re_hbm.at[pl.ds(base, tpw)], re_v)
        pltpu.sync_copy(ro_hbm.at[pl.ds(base, tpw)], ro_v)
        pltpu.sync_copy(ge_hbm.at[pl.ds(base, tpw)], ge_v)
        pltpu.sync_copy(go_hbm.at[pl.ds(base, tpw)], go_v)
        bufa = (a0, a1)
        bufb = (b0, b1)
        sems_a = (sa0, sa1)
        sems_b = (sb0, sb1)
        sems_s = (ss0, ss1)

        def fire(c):
            s = c % 2
            return (
                pltpu.async_copy(
                    ow_hbm.at[re_v.at[pl.ds(c * ch, ch)]], bufa[s], sems_a[s]
                ),
                pltpu.async_copy(
                    ow_hbm.at[ro_v.at[pl.ds(c * ch, ch)]], bufb[s], sems_b[s]
                ),
            )

        pend = fire(0)
        stores = [None, None]
        for c in range(nch):
            s = c % 2
            if c + 1 < nch:
                if stores[(c + 1) % 2] is not None:
                    stores[(c + 1) % 2].wait()
                nxt = fire(c + 1)
            pend[0].wait()
            pend[1].wait()
            if c + 1 < nch:
                pend = nxt
            av = bufa[s]
            bv = bufb[s]

            def row_body(r, _):
                ga = ge_v[c * ch + r]
                gb = go_v[c * ch + r]

                def lane_body(l, _):
                    o = l * 4 * nl
                    for u in range(4):
                        q = o + u * nl
                        av[r, pl.ds(q, nl)] = (
                            av[r, pl.ds(q, nl)] * ga + bv[r, pl.ds(q, nl)] * gb
                        )
                    return 0

                lax.fori_loop(0, nlc // 4, lane_body, 0)
                return 0

            lax.fori_loop(0, ch, row_body, 0)
            stores[s] = pltpu.async_copy(
                av, y_hbm.at[pl.ds(base + c * ch, ch)], sems_s[s]
            )
        for s in (0, 1):
            if stores[s] is not None:
                stores[s].wait()

    return k(out_w, re, ro, ge, go)


@jax.jit
def kernel(x, w_router, w_in, w_out, bias):
    bsz, length, d = x.shape
    e_num = w_router.shape[1]
    k = 2
    xf = x.reshape(-1, d)
    t = xf.shape[0]

    top_k_indices, ge_x, go_x, rankp, stats = _router(xf, w_router)
    probs_sum = stats[0, :e_num]
    freq = stats[1, :e_num]
    lsesq = stats[2, 0]
    counts = stats[3, :e_num].astype(jnp.int32)
    probs_normalized = probs_sum / jnp.sum(probs_sum)
    freq_normalized = freq / jnp.sum(freq)
    switchloss = e_num * (probs_normalized * freq_normalized).sum()
    zloss = lsesq / t
    loss = switchloss + 0.1 * zloss

    flat_experts = top_k_indices.reshape(-1)
    ends = jnp.cumsum(counts).astype(jnp.int32)
    starts = ends - counts
    rank = (starts[flat_experts] + rankp.reshape(-1)).reshape(t, k)
    nw = 32
    ch_d = min(32, t // nw)
    re3 = rank[:, 0].reshape(nw, -1, ch_d)
    ro3 = rank[:, 1].reshape(nw, -1, ch_d)

    x_sorted = _sc_dispatch_scatter(xf, re3, ro3)
    out_w = _grouped_ffn(x_sorted, w_in, w_out, bias, starts, ends)

    y = _sc_combine(out_w, rank[:, 0], rank[:, 1], ge_x, go_x)
    y = y.reshape(bsz, length, d)
    return (y, loss)
